# baseline (speedup 1.0000x reference)
import functools

import jax
import jax.numpy as jnp
from jax import lax
from jax.experimental import pallas as pl
from jax.experimental.pallas import tpu as pltpu
from jax.experimental.pallas import tpu_sc as plsc


def kernel(x, table):
    B, T = x.shape
    _, D = table.shape

    info = plsc.get_sparse_core_info()
    NC, NS = info.num_cores, info.num_subcores
    NW = NC * NS
    rows_per_w = T // NW
    CH = 32
    n_chunks = rows_per_w // CH

    mesh = plsc.VectorSubcoreMesh(core_axis_name="c", subcore_axis_name="s")

    @functools.partial(
        pl.kernel,
        out_type=jax.ShapeDtypeStruct((B, T, D), jnp.float32),
        mesh=mesh,
        scratch_types=[
            pltpu.VMEM((CH, D), jnp.float32),
            pltpu.VMEM((CH, D), jnp.float32),
            pltpu.SemaphoreType.DMA,
            pltpu.SemaphoreType.DMA,
        ],
    )
    def sc_copy(table_hbm, out_hbm, buf0, buf1, sem_r, sem_w):
        wid = lax.axis_index("s") * NC + lax.axis_index("c")
        base = wid * rows_per_w
        bufs = (buf0, buf1)

        def start_read(c):
            return pltpu.async_copy(
                table_hbm.at[pl.ds(base + c * CH, CH)], bufs[c % 2], sem_r
            )

        reads = {0: start_read(0)}
        writes = []
        for c in range(n_chunks):
            reads.pop(c).wait()
            for b in range(B):
                writes.append(
                    pltpu.async_copy(
                        bufs[c % 2], out_hbm.at[b, pl.ds(base + c * CH, CH), :], sem_w
                    )
                )
            if c + 1 < n_chunks:
                while len(writes) > B:
                    writes.pop(0).wait()
                reads[c + 1] = start_read(c + 1)
        for w in writes:
            w.wait()

    return sc_copy(table)


# speedup vs baseline: 1.0267x; 1.0267x over previous
import functools

import jax
import jax.numpy as jnp
from jax import lax
from jax.experimental import pallas as pl
from jax.experimental.pallas import tpu as pltpu
from jax.experimental.pallas import tpu_sc as plsc


def kernel(x, table):
    B, T = x.shape
    _, D = table.shape

    info = plsc.get_sparse_core_info()
    NC, NS = info.num_cores, info.num_subcores
    NW = NC * NS
    rows_per_w = T // NW
    CH = 64
    n_chunks = rows_per_w // CH

    mesh = plsc.VectorSubcoreMesh(core_axis_name="c", subcore_axis_name="s")

    @functools.partial(
        pl.kernel,
        out_type=jax.ShapeDtypeStruct((B, T, D), jnp.float32),
        mesh=mesh,
        scratch_types=[
            pltpu.VMEM((CH, D), jnp.float32),
            pltpu.SemaphoreType.DMA,
        ],
    )
    def sc_copy(table_hbm, out_hbm, buf, sem_w):
        wid = lax.axis_index("s") * NC + lax.axis_index("c")
        base = wid * rows_per_w
        for c in range(n_chunks):
            row0 = base + c * CH
            pltpu.sync_copy(table_hbm.at[pl.ds(row0, CH)], buf)
            writes = [
                pltpu.async_copy(buf, out_hbm.at[b, pl.ds(row0, CH), :], sem_w)
                for b in range(B)
            ]
            for w in writes:
                w.wait()

    return sc_copy(table)
